# Initial kernel scaffold; baseline (speedup 1.0000x reference)
import jax
import jax.numpy as jnp
from jax.experimental import pallas as pl
from jax.experimental.pallas import tpu as pltpu

_B = 4096
_T = 750
_WIN = 11
_SIGMA = 1.0
_E_THETA = 0.1
_E_G = 1.0
_E_ALPHA = 1.0
_INV11 = 2979
_TC = 128
_G = 6
_PW = (_G + 1) * _TC
_AEH = 4112


def _loss_body(p4a_ref, p4b_ref, p3a_ref, p3b_ref, ae_ref, out_ref,
               s4_ref, s3_ref):
    g = pl.program_id(0)

    s4_ref[:, :_TC] = p4a_ref[...]
    s4_ref[:, _TC:] = p4b_ref[...]
    s3_ref[:, :_TC] = p3a_ref[...]
    s3_ref[:, _TC:] = p3b_ref[...]

    ac = s4_ref[:, 6:6 + _TC]
    a2c = s3_ref[:, 6:6 + _TC]

    lane = jax.lax.broadcasted_iota(jnp.int32, (1, _TC), 1) + g * _TC
    valid = lane < _T
    ns_bias = jnp.where(valid, 0.0, jnp.float32(1e9))

    inv_two_sigma2 = jnp.float32(-0.5 / (_SIGMA * _SIGMA))

    ens = []
    for j in range(_WIN):
        d = ac - s4_ref[:, j:j + _TC]
        nsj = jnp.sum(d * d, axis=0, keepdims=True) + ns_bias
        ens.append(jnp.exp(inv_two_sigma2 * nsj))

    ens_max = ens[0]
    for j in range(1, _WIN):
        ens_max = jnp.maximum(ens_max, ens[j])
    any_live = jnp.max(ens_max) > 0.0

    delta = ac - a2c
    base = jnp.sum(delta * delta, axis=0, keepdims=True) * jnp.float32(_E_THETA)
    out_ref[...] = jnp.where(valid, base, 0.0).reshape(1, 1, _TC)

    @pl.when(any_live)
    def _():
        mw = ae_ref[0:_B, :] - s4_ref[:, 0:_TC]
        for j in range(1, _WIN):
            mw = jnp.maximum(mw, ae_ref[j:j + _B, :] - s4_ref[:, j:j + _TC])
        eg = jnp.exp(inv_two_sigma2 * jnp.float32(_E_G) * mw * mw)
        acc = jnp.minimum(ens[0], eg) * jnp.abs(a2c - s3_ref[:, 0:_TC])
        for j in range(1, _WIN):
            acc = acc + jnp.minimum(ens[j], eg) * jnp.abs(
                a2c - s3_ref[:, j:j + _TC])
        part = jnp.sum(acc, axis=0, keepdims=True)
        out_ref[...] += jnp.where(valid, part, 0.0).reshape(1, 1, _TC)


def _pad_like_ref(x):
    b = x.shape[0]
    front = jnp.tile(x[:, 0], 6).reshape(b, 6)
    back = jnp.tile(x[:, -1], 6).reshape(b, 6)
    return jnp.concatenate([front, x, back[:, 1:]], axis=1)


def kernel(actioness, actioness_2):
    sigma = (_INV11 * jnp.arange(_B, dtype=jnp.int32)) % _B

    p4 = _pad_like_ref(actioness)[sigma]
    p3 = _pad_like_ref(actioness_2)[sigma]
    p4 = jnp.pad(p4, ((0, 0), (0, _PW - p4.shape[1])))
    p3 = jnp.pad(p3, ((0, 0), (0, _PW - p3.shape[1])))

    ae = jnp.concatenate([actioness, actioness[:_WIN - 1]], axis=0)
    ae = jnp.pad(ae, ((0, _AEH - ae.shape[0]), (0, _G * _TC - _T)))

    col = pl.BlockSpec((_B, _TC), lambda i: (0, i))
    col_next = pl.BlockSpec((_B, _TC), lambda i: (0, i + 1))
    ae_spec = pl.BlockSpec((_AEH, _TC), lambda i: (0, i))

    partials = pl.pallas_call(
        _loss_body,
        grid=(_G,),
        in_specs=[col, col_next, col, col_next, ae_spec],
        out_specs=pl.BlockSpec((1, 1, _TC), lambda i: (i, 0, 0)),
        out_shape=jax.ShapeDtypeStruct((_G, 1, _TC), jnp.float32),
        scratch_shapes=[
            pltpu.VMEM((_B, 2 * _TC), jnp.float32),
            pltpu.VMEM((_B, 2 * _TC), jnp.float32),
        ],
        compiler_params=pltpu.CompilerParams(
            dimension_semantics=("parallel",),
            vmem_limit_bytes=48 * 1024 * 1024,
        ),
        name="act_eloss_v3",
    )(p4, p4, p3, p3, ae)

    return jnp.float32(_E_ALPHA / _B) * jnp.sum(partials)


# speedup vs baseline: 3.5273x; 3.5273x over previous
import jax
import jax.numpy as jnp
from jax.experimental import pallas as pl
from jax.experimental.pallas import tpu as pltpu

_B = 4096
_T = 750
_WIN = 11
_SIGMA = 1.0
_E_THETA = 0.1
_E_G = 1.0
_E_ALPHA = 1.0
_INV11 = 2979
_TC = 128
_G = 6
_PW = (_G + 1) * _TC
_AEH = 4112


_CB = 128


def _loss_body(p4a_ref, p4b_ref, p3a_ref, p3b_ref, ae_ref, out_ref):
    g = pl.program_id(0)

    def win(aref, bref, r, j):
        rows = pl.ds(r, _CB)
        if j == 0:
            return aref[rows, :]
        return jnp.concatenate([aref[rows, j:], bref[rows, :j]], axis=1)

    def fold8(x):
        return jnp.sum(x.reshape(_CB // 8, 8, _TC), axis=0)

    lane = jax.lax.broadcasted_iota(jnp.int32, (1, _TC), 1) + g * _TC
    valid = lane < _T
    ns_bias = jnp.where(valid, 0.0, jnp.float32(1e9))

    inv_two_sigma2 = jnp.float32(-0.5 / (_SIGMA * _SIGMA))

    def ns_chunk(i, carry):
        r = i * _CB
        ac = win(p4a_ref, p4b_ref, r, 6)
        new = [None] * (_WIN + 1)
        for j in range(_WIN):
            d = ac - win(p4a_ref, p4b_ref, r, j)
            new[j] = carry[j] + fold8(d * d)
        dd = ac - win(p3a_ref, p3b_ref, r, 6)
        new[_WIN] = carry[_WIN] + fold8(dd * dd)
        return tuple(new)

    zeros = jnp.zeros((8, _TC), jnp.float32)
    *ns_acc, th_acc = jax.lax.fori_loop(
        0, _B // _CB, ns_chunk, (zeros,) * (_WIN + 1))

    ens = [jnp.exp(inv_two_sigma2 *
                   (jnp.sum(ns_acc[j], axis=0, keepdims=True) + ns_bias))
           for j in range(_WIN)]
    ens_max = ens[0]
    for j in range(1, _WIN):
        ens_max = jnp.maximum(ens_max, ens[j])
    any_live = jnp.max(ens_max) > 0.0

    base = jnp.sum(th_acc, axis=0, keepdims=True) * jnp.float32(_E_THETA)
    out_ref[...] = jnp.where(valid, base, 0.0).reshape(1, 1, _TC)

    @pl.when(any_live)
    def _():
        def l1_chunk(i, tot):
            r = i * _CB
            mw = ae_ref[pl.ds(r, _CB), :] - win(p4a_ref, p4b_ref, r, 0)
            for j in range(1, _WIN):
                mw = jnp.maximum(
                    mw, ae_ref[pl.ds(r + j, _CB), :]
                    - win(p4a_ref, p4b_ref, r, j))
            eg = jnp.exp(inv_two_sigma2 * jnp.float32(_E_G) * mw * mw)
            a2 = win(p3a_ref, p3b_ref, r, 6)
            acc = jnp.minimum(ens[0], eg) * jnp.abs(
                a2 - win(p3a_ref, p3b_ref, r, 0))
            for j in range(1, _WIN):
                acc = acc + jnp.minimum(ens[j], eg) * jnp.abs(
                    a2 - win(p3a_ref, p3b_ref, r, j))
            return tot + fold8(acc)

        tot = jax.lax.fori_loop(0, _B // _CB, l1_chunk,
                                jnp.zeros((8, _TC), jnp.float32))
        part = jnp.sum(tot, axis=0, keepdims=True)
        out_ref[...] += jnp.where(valid, part, 0.0).reshape(1, 1, _TC)


def _pad_like_ref(x):
    b = x.shape[0]
    front = jnp.tile(x[:, 0], 6).reshape(b, 6)
    back = jnp.tile(x[:, -1], 6).reshape(b, 6)
    return jnp.concatenate([front, x, back[:, 1:]], axis=1)


def kernel(actioness, actioness_2):
    sigma = (_INV11 * jnp.arange(_B, dtype=jnp.int32)) % _B

    p4 = _pad_like_ref(actioness)[sigma]
    p3 = _pad_like_ref(actioness_2)[sigma]
    p4 = jnp.pad(p4, ((0, 0), (0, _PW - p4.shape[1])))
    p3 = jnp.pad(p3, ((0, 0), (0, _PW - p3.shape[1])))

    ae = jnp.concatenate([actioness, actioness[:_WIN - 1]], axis=0)
    ae = jnp.pad(ae, ((0, _AEH - ae.shape[0]), (0, _G * _TC - _T)))

    col = pl.BlockSpec((_B, _TC), lambda i: (0, i))
    col_next = pl.BlockSpec((_B, _TC), lambda i: (0, i + 1))
    ae_spec = pl.BlockSpec((_AEH, _TC), lambda i: (0, i))

    partials = pl.pallas_call(
        _loss_body,
        grid=(_G,),
        in_specs=[col, col_next, col, col_next, ae_spec],
        out_specs=pl.BlockSpec((1, 1, _TC), lambda i: (i, 0, 0)),
        out_shape=jax.ShapeDtypeStruct((_G, 1, _TC), jnp.float32),
        compiler_params=pltpu.CompilerParams(
            dimension_semantics=("parallel",),
            vmem_limit_bytes=48 * 1024 * 1024,
        ),
        name="act_eloss_v3",
    )(p4, p4, p3, p3, ae)

    return jnp.float32(_E_ALPHA / _B) * jnp.sum(partials)


# speedup vs baseline: 3.8189x; 1.0827x over previous
import jax
import jax.numpy as jnp
from jax.experimental import pallas as pl
from jax.experimental.pallas import tpu as pltpu

_B = 4096
_T = 750
_WIN = 11
_SIGMA = 1.0
_E_THETA = 0.1
_E_G = 1.0
_E_ALPHA = 1.0
_TC = 128
_G = 6
_PW = (_G + 1) * _TC
_CB = 128
_AEH = 5376


def _loss_body(p4a_ref, p4b_ref, p3a_ref, p3b_ref, ae_ref, out_ref):
    g = pl.program_id(0)

    def win(aref, bref, r, j):
        rows = pl.ds(r, _CB)
        if j == 0:
            return aref[rows, :]
        return jnp.concatenate([aref[rows, j:], bref[rows, :j]], axis=1)

    def fold8(x):
        return jnp.sum(x.reshape(_CB // 8, 8, _TC), axis=0)

    lane = jax.lax.broadcasted_iota(jnp.int32, (1, _TC), 1) + g * _TC
    valid = lane < _T
    ns_bias = jnp.where(valid, 0.0, jnp.float32(1e9))

    inv_two_sigma2 = jnp.float32(-0.5 / (_SIGMA * _SIGMA))

    def ns_chunk(i, carry):
        r = i * _CB
        ac = win(p4a_ref, p4b_ref, r, 6)
        new = [None] * (_WIN + 1)
        for j in range(_WIN):
            d = ac - win(p4a_ref, p4b_ref, r, j)
            new[j] = carry[j] + fold8(d * d)
        dd = ac - win(p3a_ref, p3b_ref, r, 6)
        new[_WIN] = carry[_WIN] + fold8(dd * dd)
        return tuple(new)

    zeros = jnp.zeros((8, _TC), jnp.float32)
    *ns_acc, th_acc = jax.lax.fori_loop(
        0, _B // _CB, ns_chunk, (zeros,) * (_WIN + 1))

    ens = [jnp.exp(inv_two_sigma2 *
                   (jnp.sum(ns_acc[j], axis=0, keepdims=True) + ns_bias))
           for j in range(_WIN)]
    ens_max = ens[0]
    for j in range(1, _WIN):
        ens_max = jnp.maximum(ens_max, ens[j])
    any_live = jnp.max(ens_max) > 0.0

    base = jnp.sum(th_acc, axis=0, keepdims=True) * jnp.float32(_E_THETA)
    out_ref[...] = jnp.where(valid, base, 0.0).reshape(1, 1, _TC)

    @pl.when(any_live)
    def _():
        def l1_chunk(i, tot):
            r = i * _CB
            s = jax.lax.rem(jnp.int32(11) * _CB * i, jnp.int32(_B))
            mw = (ae_ref[pl.Slice(s, _CB, _WIN), :]
                  - win(p4a_ref, p4b_ref, r, 0))
            for j in range(1, _WIN):
                mw = jnp.maximum(
                    mw, ae_ref[pl.Slice(s + j, _CB, _WIN), :]
                    - win(p4a_ref, p4b_ref, r, j))
            eg = jnp.exp(inv_two_sigma2 * jnp.float32(_E_G) * mw * mw)
            a2 = win(p3a_ref, p3b_ref, r, 6)
            acc = jnp.minimum(ens[0], eg) * jnp.abs(
                a2 - win(p3a_ref, p3b_ref, r, 0))
            for j in range(1, _WIN):
                acc = acc + jnp.minimum(ens[j], eg) * jnp.abs(
                    a2 - win(p3a_ref, p3b_ref, r, j))
            return tot + fold8(acc)

        tot = jax.lax.fori_loop(0, _B // _CB, l1_chunk,
                                jnp.zeros((8, _TC), jnp.float32))
        part = jnp.sum(tot, axis=0, keepdims=True)
        out_ref[...] += jnp.where(valid, part, 0.0).reshape(1, 1, _TC)


def _pad_like_ref(x):
    b = x.shape[0]
    front = jnp.tile(x[:, 0], 6).reshape(b, 6)
    back = jnp.tile(x[:, -1], 6).reshape(b, 6)
    zpad = jnp.zeros((b, _PW - (_T + _WIN)), x.dtype)
    return jnp.concatenate([front, x, back[:, 1:], zpad], axis=1)


def kernel(actioness, actioness_2):
    p4 = _pad_like_ref(actioness)
    p3 = _pad_like_ref(actioness_2)

    ae = jnp.concatenate([actioness, actioness[:_AEH - _B]], axis=0)
    ae = jnp.pad(ae, ((0, 0), (0, _G * _TC - _T)))

    col = pl.BlockSpec((_B, _TC), lambda i: (0, i))
    col_next = pl.BlockSpec((_B, _TC), lambda i: (0, i + 1))
    ae_spec = pl.BlockSpec((_AEH, _TC), lambda i: (0, i))

    partials = pl.pallas_call(
        _loss_body,
        grid=(_G,),
        in_specs=[col, col_next, col, col_next, ae_spec],
        out_specs=pl.BlockSpec((1, 1, _TC), lambda i: (i, 0, 0)),
        out_shape=jax.ShapeDtypeStruct((_G, 1, _TC), jnp.float32),
        compiler_params=pltpu.CompilerParams(
            dimension_semantics=("parallel",),
            vmem_limit_bytes=48 * 1024 * 1024,
        ),
        name="act_eloss_v3",
    )(p4, p4, p3, p3, ae)

    return jnp.float32(_E_ALPHA / _B) * jnp.sum(partials)
